# Initial kernel scaffold; baseline (speedup 1.0000x reference)
import jax
import jax.numpy as jnp
from jax.experimental import pallas as pl


def kernel(pred_obj_logits, pred_verb_logits, pred_sub_boxes, pred_obj_boxes, target_sizes, correct_mat):
    raise NotImplementedError("write your pallas kernel here")


# speedup vs baseline: 23.2587x; 23.2587x over previous
import jax
import jax.numpy as jnp
from jax.experimental import pallas as pl
from jax.experimental.pallas import tpu as pltpu

_NMS_THRESH = 0.7


def _hoi_nms_kernel(obj_ref, verb_ref, subb_ref, objb_ref, scale_ref, cmt_ref,
                    hoi_ref, lab_ref, subo_ref, objo_ref, keep_ref):
    obj_logits = obj_ref[0]
    verb_logits = verb_ref[0]
    Q, NO = obj_logits.shape

    sig = jax.nn.sigmoid(obj_logits)
    obj_scores = jnp.max(sig, axis=-1, keepdims=True)
    mx = jnp.max(obj_logits, axis=-1, keepdims=True)
    e = jnp.exp(obj_logits - mx)
    emx = jnp.max(e, axis=-1, keepdims=True)
    col = jax.lax.broadcasted_iota(jnp.int32, (Q, NO), 1)
    labels = jnp.min(jnp.where(e == emx, col, NO), axis=-1, keepdims=True)
    lab_ref[0] = labels

    verb_scores = jax.nn.sigmoid(verb_logits)
    onehot = (col == labels).astype(jnp.float32)
    masks = jnp.dot(onehot, cmt_ref[...], preferred_element_type=jnp.float32)
    hoi = verb_scores * obj_scores * masks
    hoi_ref[0] = hoi

    scale = scale_ref[0]

    def to_xyxy(b):
        xc, yc, w, h = b[:, 0:1], b[:, 1:2], b[:, 2:3], b[:, 3:4]
        return jnp.concatenate(
            [xc - 0.5 * w, yc - 0.5 * h, xc + 0.5 * w, yc + 0.5 * h], axis=1)

    sb = to_xyxy(subb_ref[0]) * scale
    ob = to_xyxy(objb_ref[0]) * scale
    subo_ref[0] = sb
    objo_ref[0] = ob

    def iou(bx):
        x1, y1, x2, y2 = bx[:, 0:1], bx[:, 1:2], bx[:, 2:3], bx[:, 3:4]
        bt = jnp.transpose(bx)
        x1r, y1r, x2r, y2r = bt[0:1], bt[1:2], bt[2:3], bt[3:4]
        area_c = (x2 - x1 + 1.0) * (y2 - y1 + 1.0)
        area_r = (x2r - x1r + 1.0) * (y2r - y1r + 1.0)
        xx1 = jnp.maximum(x1, x1r)
        yy1 = jnp.maximum(y1, y1r)
        xx2 = jnp.minimum(x2, x2r)
        yy2 = jnp.minimum(y2, y2r)
        w = jnp.maximum(0.0, xx2 - xx1 + 1.0)
        h = jnp.maximum(0.0, yy2 - yy1 + 1.0)
        inter = w * h
        union = area_c + area_r - inter
        return inter / union

    ovr = iou(sb) * jnp.sqrt(iou(ob))

    ms_c = jnp.max(hoi, axis=-1, keepdims=True)
    ms_r = jnp.transpose(ms_c)
    lab_r = jnp.transpose(labels)
    row_i = jax.lax.broadcasted_iota(jnp.int32, (Q, Q), 0)
    col_i = jax.lax.broadcasted_iota(jnp.int32, (Q, Q), 1)
    prec = (ms_c > ms_r) | ((ms_c == ms_r) & (row_i < col_i))
    sup = (ovr > _NMS_THRESH) & (labels == lab_r) & prec
    m = sup.astype(jnp.float32)

    def cond(c):
        return c[1]

    def body(c):
        keep, _ = c
        cnt = jnp.sum(m * keep, axis=0, keepdims=True)
        new_r = (cnt < 0.5).astype(jnp.float32)
        new_c = jnp.transpose(new_r)
        return new_c, jnp.any(new_c != keep)

    keep0 = jnp.ones((Q, 1), jnp.float32)
    keep, _ = jax.lax.while_loop(cond, body, (keep0, jnp.bool_(True)))
    keep_ref[0] = keep


def kernel(pred_obj_logits, pred_verb_logits, pred_sub_boxes, pred_obj_boxes,
           target_sizes, correct_mat):
    B, Q, NO = pred_obj_logits.shape
    NV = pred_verb_logits.shape[-1]
    cm = jnp.concatenate([correct_mat, jnp.ones((NV, 1), correct_mat.dtype)],
                         axis=1)
    cmt = cm.T
    img_h = target_sizes[:, 0].astype(jnp.float32)
    img_w = target_sizes[:, 1].astype(jnp.float32)
    scale = jnp.stack([img_w, img_h, img_w, img_h], axis=1).reshape(B, 1, 4)

    hoi, labels, sb, ob, keepf = pl.pallas_call(
        _hoi_nms_kernel,
        grid=(B,),
        in_specs=[
            pl.BlockSpec((1, Q, NO), lambda b: (b, 0, 0)),
            pl.BlockSpec((1, Q, NV), lambda b: (b, 0, 0)),
            pl.BlockSpec((1, Q, 4), lambda b: (b, 0, 0)),
            pl.BlockSpec((1, Q, 4), lambda b: (b, 0, 0)),
            pl.BlockSpec((1, 1, 4), lambda b: (b, 0, 0)),
            pl.BlockSpec((NO, NV), lambda b: (0, 0)),
        ],
        out_specs=[
            pl.BlockSpec((1, Q, NV), lambda b: (b, 0, 0)),
            pl.BlockSpec((1, Q, 1), lambda b: (b, 0, 0)),
            pl.BlockSpec((1, Q, 4), lambda b: (b, 0, 0)),
            pl.BlockSpec((1, Q, 4), lambda b: (b, 0, 0)),
            pl.BlockSpec((1, Q, 1), lambda b: (b, 0, 0)),
        ],
        out_shape=[
            jax.ShapeDtypeStruct((B, Q, NV), jnp.float32),
            jax.ShapeDtypeStruct((B, Q, 1), jnp.int32),
            jax.ShapeDtypeStruct((B, Q, 4), jnp.float32),
            jax.ShapeDtypeStruct((B, Q, 4), jnp.float32),
            jax.ShapeDtypeStruct((B, Q, 1), jnp.float32),
        ],
    )(pred_obj_logits, pred_verb_logits, pred_sub_boxes, pred_obj_boxes,
      scale, cmt)

    return hoi, labels[..., 0], sb, ob, keepf[..., 0] > 0.5


# speedup vs baseline: 23.4984x; 1.0103x over previous
import jax
import jax.numpy as jnp
from jax.experimental import pallas as pl
from jax.experimental.pallas import tpu as pltpu

_NMS_THRESH = 0.7


def _hoi_nms_kernel(obj_ref, verb_ref, subb_ref, objb_ref, scale_ref, cmt_ref,
                    hoi_ref, lab_ref, subo_ref, objo_ref, keep_ref):
    obj_logits = obj_ref[0]
    verb_logits = verb_ref[0]
    Q, NO = obj_logits.shape

    sig = jax.nn.sigmoid(obj_logits)
    obj_scores = jnp.max(sig, axis=-1, keepdims=True)
    mx = jnp.max(obj_logits, axis=-1, keepdims=True)
    e = jnp.exp(obj_logits - mx)
    emx = jnp.max(e, axis=-1, keepdims=True)
    col = jax.lax.broadcasted_iota(jnp.int32, (Q, NO), 1)
    labels = jnp.min(jnp.where(e == emx, col, NO), axis=-1, keepdims=True)
    lab_ref[0] = labels

    verb_scores = jax.nn.sigmoid(verb_logits)
    onehot = (col == labels).astype(jnp.float32)
    masks = jnp.dot(onehot, cmt_ref[...], preferred_element_type=jnp.float32)
    hoi = verb_scores * obj_scores * masks
    hoi_ref[0] = hoi

    scale = scale_ref[0]

    def to_xyxy(b):
        xc, yc, w, h = b[:, 0:1], b[:, 1:2], b[:, 2:3], b[:, 3:4]
        return jnp.concatenate(
            [xc - 0.5 * w, yc - 0.5 * h, xc + 0.5 * w, yc + 0.5 * h], axis=1)

    sb = to_xyxy(subb_ref[0]) * scale
    ob = to_xyxy(objb_ref[0]) * scale
    subo_ref[0] = sb
    objo_ref[0] = ob

    def inter_union(bx):
        x1, y1, x2, y2 = bx[:, 0:1], bx[:, 1:2], bx[:, 2:3], bx[:, 3:4]
        bt = jnp.transpose(bx)
        x1r, y1r, x2r, y2r = bt[0:1], bt[1:2], bt[2:3], bt[3:4]
        area_c = (x2 - x1 + 1.0) * (y2 - y1 + 1.0)
        area_r = (x2r - x1r + 1.0) * (y2r - y1r + 1.0)
        xx1 = jnp.maximum(x1, x1r)
        yy1 = jnp.maximum(y1, y1r)
        xx2 = jnp.minimum(x2, x2r)
        yy2 = jnp.minimum(y2, y2r)
        w = jnp.maximum(0.0, xx2 - xx1 + 1.0)
        h = jnp.maximum(0.0, yy2 - yy1 + 1.0)
        inter = w * h
        union = area_c + area_r - inter
        return inter, union

    i_s, u_s = inter_union(sb)
    i_o, u_o = inter_union(ob)
    over_t = (i_s * i_s) * i_o > ((_NMS_THRESH * _NMS_THRESH) * u_s) * (u_s * u_o)

    ms_c = jnp.max(hoi, axis=-1, keepdims=True)
    ms_r = jnp.transpose(ms_c)
    lab_r = jnp.transpose(labels)
    row_i = jax.lax.broadcasted_iota(jnp.int32, (Q, Q), 0)
    col_i = jax.lax.broadcasted_iota(jnp.int32, (Q, Q), 1)
    prec = (ms_c > ms_r) | ((ms_c == ms_r) & (row_i < col_i))
    sup = over_t & (labels == lab_r) & prec
    m = sup.astype(jnp.float32)

    def cond(c):
        return c[1]

    def body(c):
        keep, _ = c
        cnt = jnp.sum(m * keep, axis=0, keepdims=True)
        new_r = (cnt < 0.5).astype(jnp.float32)
        new_c = jnp.transpose(new_r)
        return new_c, jnp.any(new_c != keep)

    keep0 = jnp.ones((Q, 1), jnp.float32)
    keep, _ = jax.lax.while_loop(cond, body, (keep0, jnp.bool_(True)))
    keep_ref[0] = keep


def kernel(pred_obj_logits, pred_verb_logits, pred_sub_boxes, pred_obj_boxes,
           target_sizes, correct_mat):
    B, Q, NO = pred_obj_logits.shape
    NV = pred_verb_logits.shape[-1]
    cm = jnp.concatenate([correct_mat, jnp.ones((NV, 1), correct_mat.dtype)],
                         axis=1)
    cmt = cm.T
    img_h = target_sizes[:, 0].astype(jnp.float32)
    img_w = target_sizes[:, 1].astype(jnp.float32)
    scale = jnp.stack([img_w, img_h, img_w, img_h], axis=1).reshape(B, 1, 4)

    hoi, labels, sb, ob, keepf = pl.pallas_call(
        _hoi_nms_kernel,
        grid=(B,),
        in_specs=[
            pl.BlockSpec((1, Q, NO), lambda b: (b, 0, 0)),
            pl.BlockSpec((1, Q, NV), lambda b: (b, 0, 0)),
            pl.BlockSpec((1, Q, 4), lambda b: (b, 0, 0)),
            pl.BlockSpec((1, Q, 4), lambda b: (b, 0, 0)),
            pl.BlockSpec((1, 1, 4), lambda b: (b, 0, 0)),
            pl.BlockSpec((NO, NV), lambda b: (0, 0)),
        ],
        out_specs=[
            pl.BlockSpec((1, Q, NV), lambda b: (b, 0, 0)),
            pl.BlockSpec((1, Q, 1), lambda b: (b, 0, 0)),
            pl.BlockSpec((1, Q, 4), lambda b: (b, 0, 0)),
            pl.BlockSpec((1, Q, 4), lambda b: (b, 0, 0)),
            pl.BlockSpec((1, Q, 1), lambda b: (b, 0, 0)),
        ],
        out_shape=[
            jax.ShapeDtypeStruct((B, Q, NV), jnp.float32),
            jax.ShapeDtypeStruct((B, Q, 1), jnp.int32),
            jax.ShapeDtypeStruct((B, Q, 4), jnp.float32),
            jax.ShapeDtypeStruct((B, Q, 4), jnp.float32),
            jax.ShapeDtypeStruct((B, Q, 1), jnp.float32),
        ],
    )(pred_obj_logits, pred_verb_logits, pred_sub_boxes, pred_obj_boxes,
      scale, cmt)

    return hoi, labels[..., 0], sb, ob, keepf[..., 0] > 0.5


# speedup vs baseline: 25.5105x; 1.0856x over previous
import jax
import jax.numpy as jnp
from jax.experimental import pallas as pl
from jax.experimental.pallas import tpu as pltpu

_NMS_THRESH = 0.7


def _hoi_nms_kernel(obj_ref, verb_ref, subb_ref, objb_ref, ts_ref, cm_ref,
                    hoi_ref, lab_ref, subo_ref, objo_ref, keep_ref):
    obj_logits = obj_ref[0]
    verb_logits = verb_ref[0]
    Q, NO = obj_logits.shape

    sig = jax.nn.sigmoid(obj_logits)
    obj_scores = jnp.max(sig, axis=-1, keepdims=True)
    mx = jnp.max(obj_logits, axis=-1, keepdims=True)
    e = jnp.exp(obj_logits - mx)
    emx = jnp.max(e, axis=-1, keepdims=True)
    col = jax.lax.broadcasted_iota(jnp.int32, (Q, NO), 1)
    labels = jnp.min(jnp.where(e == emx, col, NO), axis=-1, keepdims=True)
    lab_ref[0] = labels

    verb_scores = jax.nn.sigmoid(verb_logits)
    onehot = (col == labels).astype(jnp.float32)
    masks = jax.lax.dot_general(
        onehot[:, :NO - 1], cm_ref[...],
        (((1,), (1,)), ((), ())), preferred_element_type=jnp.float32)
    masks = masks + (labels == NO - 1).astype(jnp.float32)
    hoi = verb_scores * obj_scores * masks
    hoi_ref[0] = hoi

    ts = ts_ref[0].astype(jnp.float32)
    scale = jnp.concatenate(
        [ts[:, 1:2], ts[:, 0:1], ts[:, 1:2], ts[:, 0:1]], axis=1)

    def to_xyxy(b):
        xc, yc, w, h = b[:, 0:1], b[:, 1:2], b[:, 2:3], b[:, 3:4]
        return jnp.concatenate(
            [xc - 0.5 * w, yc - 0.5 * h, xc + 0.5 * w, yc + 0.5 * h], axis=1)

    sb = to_xyxy(subb_ref[0]) * scale
    ob = to_xyxy(objb_ref[0]) * scale
    subo_ref[0] = sb
    objo_ref[0] = ob

    def inter_union(bx):
        x1, y1 = bx[:, 0:1], bx[:, 1:2]
        x2p, y2p = bx[:, 2:3] + 1.0, bx[:, 3:4] + 1.0
        area_c = (x2p - x1) * (y2p - y1)
        bt = jnp.transpose(bx)
        x1r, y1r = bt[0:1], bt[1:2]
        x2pr, y2pr = bt[2:3] + 1.0, bt[3:4] + 1.0
        area_r = (x2pr - x1r) * (y2pr - y1r)
        w = jnp.maximum(0.0, jnp.minimum(x2p, x2pr) - jnp.maximum(x1, x1r))
        h = jnp.maximum(0.0, jnp.minimum(y2p, y2pr) - jnp.maximum(y1, y1r))
        inter = w * h
        union = (area_c + area_r) - inter
        return inter, union

    i_s, u_s = inter_union(sb)
    i_o, u_o = inter_union(ob)
    over_t = (i_s * i_s) * i_o > ((_NMS_THRESH * _NMS_THRESH) * u_s) * (u_s * u_o)

    ms_c = jnp.max(hoi, axis=-1, keepdims=True)
    ms_r = jnp.transpose(ms_c)
    lab_r = jnp.transpose(labels)
    row_i = jax.lax.broadcasted_iota(jnp.int32, (Q, Q), 0)
    col_i = jax.lax.broadcasted_iota(jnp.int32, (Q, Q), 1)
    prec = (ms_c > ms_r) | ((ms_c == ms_r) & (row_i < col_i))
    sup = over_t & (labels == lab_r) & prec

    def cond(c):
        return c[1]

    def body(c):
        keep, _ = c
        cnt = jnp.sum(jnp.where(sup, keep, 0.0), axis=0, keepdims=True)
        new_r = (cnt < 0.5).astype(jnp.float32)
        new_c = jnp.transpose(new_r)
        return new_c, jnp.any(new_c != keep)

    keep0 = jnp.ones((Q, 1), jnp.float32)
    keep, _ = jax.lax.while_loop(cond, body, (keep0, jnp.bool_(True)))
    keep_ref[0] = keep


def kernel(pred_obj_logits, pred_verb_logits, pred_sub_boxes, pred_obj_boxes,
           target_sizes, correct_mat):
    B, Q, NO = pred_obj_logits.shape
    NV = pred_verb_logits.shape[-1]

    hoi, labels, sb, ob, keepf = pl.pallas_call(
        _hoi_nms_kernel,
        grid=(B,),
        in_specs=[
            pl.BlockSpec((1, Q, NO), lambda b: (b, 0, 0)),
            pl.BlockSpec((1, Q, NV), lambda b: (b, 0, 0)),
            pl.BlockSpec((1, Q, 4), lambda b: (b, 0, 0)),
            pl.BlockSpec((1, Q, 4), lambda b: (b, 0, 0)),
            pl.BlockSpec((1, 1, 2), lambda b: (b, 0, 0)),
            pl.BlockSpec((NV, NO - 1), lambda b: (0, 0)),
        ],
        out_specs=[
            pl.BlockSpec((1, Q, NV), lambda b: (b, 0, 0)),
            pl.BlockSpec((1, Q, 1), lambda b: (b, 0, 0)),
            pl.BlockSpec((1, Q, 4), lambda b: (b, 0, 0)),
            pl.BlockSpec((1, Q, 4), lambda b: (b, 0, 0)),
            pl.BlockSpec((1, Q, 1), lambda b: (b, 0, 0)),
        ],
        out_shape=[
            jax.ShapeDtypeStruct((B, Q, NV), jnp.float32),
            jax.ShapeDtypeStruct((B, Q, 1), jnp.int32),
            jax.ShapeDtypeStruct((B, Q, 4), jnp.float32),
            jax.ShapeDtypeStruct((B, Q, 4), jnp.float32),
            jax.ShapeDtypeStruct((B, Q, 1), jnp.float32),
        ],
    )(pred_obj_logits, pred_verb_logits, pred_sub_boxes, pred_obj_boxes,
      target_sizes.reshape(B, 1, 2), correct_mat)

    return hoi, labels[..., 0], sb, ob, keepf[..., 0] > 0.5


# speedup vs baseline: 25.5640x; 1.0021x over previous
import jax
import jax.numpy as jnp
from jax.experimental import pallas as pl
from jax.experimental.pallas import tpu as pltpu

_NMS_THRESH = 0.7


def _hoi_nms_kernel(obj_ref, verb_ref, subb_ref, objb_ref, ts_ref, cm_ref,
                    hoi_ref, lab_ref, subo_ref, objo_ref, keep_ref):
    b = pl.program_id(0)
    obj_logits = obj_ref[0]
    verb_logits = verb_ref[0]
    Q, NO = obj_logits.shape

    sig = jax.nn.sigmoid(obj_logits)
    obj_scores = jnp.max(sig, axis=-1, keepdims=True)
    mx = jnp.max(obj_logits, axis=-1, keepdims=True)
    e = jnp.exp(obj_logits - mx)
    emx = jnp.max(e, axis=-1, keepdims=True)
    col = jax.lax.broadcasted_iota(jnp.int32, (Q, NO), 1)
    labels = jnp.min(jnp.where(e == emx, col, NO), axis=-1, keepdims=True)
    lab_r = jnp.transpose(labels)
    lab_ref[pl.ds(b, 1), :] = lab_r

    verb_scores = jax.nn.sigmoid(verb_logits)
    onehot = (col == labels).astype(jnp.float32)
    masks = jax.lax.dot_general(
        onehot[:, :NO - 1], cm_ref[...],
        (((1,), (1,)), ((), ())), preferred_element_type=jnp.float32)
    masks = masks + (labels == NO - 1).astype(jnp.float32)
    hoi = verb_scores * obj_scores * masks
    hoi_ref[0] = hoi

    ts = ts_ref[pl.ds(b, 1), :].astype(jnp.float32)
    scale = jnp.concatenate(
        [ts[:, 1:2], ts[:, 0:1], ts[:, 1:2], ts[:, 0:1]], axis=1)

    def to_xyxy(b):
        xc, yc, w, h = b[:, 0:1], b[:, 1:2], b[:, 2:3], b[:, 3:4]
        return jnp.concatenate(
            [xc - 0.5 * w, yc - 0.5 * h, xc + 0.5 * w, yc + 0.5 * h], axis=1)

    sb = to_xyxy(subb_ref[0]) * scale
    ob = to_xyxy(objb_ref[0]) * scale
    subo_ref[0] = sb
    objo_ref[0] = ob

    def inter_union(bx):
        x1, y1 = bx[:, 0:1], bx[:, 1:2]
        x2p, y2p = bx[:, 2:3] + 1.0, bx[:, 3:4] + 1.0
        area_c = (x2p - x1) * (y2p - y1)
        bt = jnp.transpose(bx)
        x1r, y1r = bt[0:1], bt[1:2]
        x2pr, y2pr = bt[2:3] + 1.0, bt[3:4] + 1.0
        area_r = (x2pr - x1r) * (y2pr - y1r)
        w = jnp.maximum(0.0, jnp.minimum(x2p, x2pr) - jnp.maximum(x1, x1r))
        h = jnp.maximum(0.0, jnp.minimum(y2p, y2pr) - jnp.maximum(y1, y1r))
        inter = w * h
        union = (area_c + area_r) - inter
        return inter, union

    i_s, u_s = inter_union(sb)
    i_o, u_o = inter_union(ob)
    over_t = (i_s * i_s) * i_o > ((_NMS_THRESH * _NMS_THRESH) * u_s) * (u_s * u_o)

    ms_c = jnp.max(hoi, axis=-1, keepdims=True)
    ms_r = jnp.transpose(ms_c)
    row_i = jax.lax.broadcasted_iota(jnp.int32, (Q, Q), 0)
    col_i = jax.lax.broadcasted_iota(jnp.int32, (Q, Q), 1)
    prec = (ms_c > ms_r) | ((ms_c == ms_r) & (row_i < col_i))
    sup = over_t & (labels == lab_r) & prec

    def cond(c):
        return c[2]

    def body(c):
        keep, _, _ = c
        cnt = jnp.sum(jnp.where(sup, keep, 0.0), axis=0, keepdims=True)
        new_r = (cnt < 0.5).astype(jnp.float32)
        new_c = jnp.transpose(new_r)
        return new_c, new_r, jnp.any(new_c != keep)

    keep0 = jnp.ones((Q, 1), jnp.float32)
    keep0_r = jnp.ones((1, Q), jnp.float32)
    _, keep_r, _ = jax.lax.while_loop(
        cond, body, (keep0, keep0_r, jnp.bool_(True)))
    keep_ref[pl.ds(b, 1), :] = keep_r > 0.5


def kernel(pred_obj_logits, pred_verb_logits, pred_sub_boxes, pred_obj_boxes,
           target_sizes, correct_mat):
    B, Q, NO = pred_obj_logits.shape
    NV = pred_verb_logits.shape[-1]

    return pl.pallas_call(
        _hoi_nms_kernel,
        grid=(B,),
        in_specs=[
            pl.BlockSpec((1, Q, NO), lambda b: (b, 0, 0)),
            pl.BlockSpec((1, Q, NV), lambda b: (b, 0, 0)),
            pl.BlockSpec((1, Q, 4), lambda b: (b, 0, 0)),
            pl.BlockSpec((1, Q, 4), lambda b: (b, 0, 0)),
            pl.BlockSpec((B, 2), lambda b: (0, 0)),
            pl.BlockSpec((NV, NO - 1), lambda b: (0, 0)),
        ],
        out_specs=[
            pl.BlockSpec((1, Q, NV), lambda b: (b, 0, 0)),
            pl.BlockSpec((B, Q), lambda b: (0, 0)),
            pl.BlockSpec((1, Q, 4), lambda b: (b, 0, 0)),
            pl.BlockSpec((1, Q, 4), lambda b: (b, 0, 0)),
            pl.BlockSpec((B, Q), lambda b: (0, 0)),
        ],
        out_shape=[
            jax.ShapeDtypeStruct((B, Q, NV), jnp.float32),
            jax.ShapeDtypeStruct((B, Q), jnp.int32),
            jax.ShapeDtypeStruct((B, Q, 4), jnp.float32),
            jax.ShapeDtypeStruct((B, Q, 4), jnp.float32),
            jax.ShapeDtypeStruct((B, Q), jnp.bool_),
        ],
    )(pred_obj_logits, pred_verb_logits, pred_sub_boxes, pred_obj_boxes,
      target_sizes, correct_mat)
